# baseline (speedup 1.0000x reference)
import functools

import jax
import jax.numpy as jnp
from jax import lax
from jax.experimental import pallas as pl
from jax.experimental.pallas import tpu as pltpu
from jax.experimental.pallas import tpu_sc as plsc

_N_NODES = 10000
_N_EDGES = 320000
_D = 128

_info = plsc.get_sparse_core_info()
_NC = _info.num_cores
_NS = _info.num_subcores
_L = _info.num_lanes
_NW = _NC * _NS
_E_PER_W = _N_EDGES // _NW


_NBLK = 1024
_N_PAD = 10 * _NBLK


def _tc_tables_body(x_ref, w12t_ref, b_ref, s_ref, t_ref):
    r = lax.dot_general(
        w12t_ref[...], x_ref[...],
        dimension_numbers=(((1,), (1,)), ((), ())),
        preferred_element_type=jnp.float32,
    )
    s_ref[...] = r[0:1, :].reshape(_NBLK) + b_ref[0]
    t_ref[...] = r[1:2, :].reshape(_NBLK)


def _make_tables(node_embedding, w12t, b):
    s, t = pl.pallas_call(
        _tc_tables_body,
        grid=(_N_PAD // _NBLK,),
        in_specs=[
            pl.BlockSpec((_NBLK, _D), lambda i: (i, 0)),
            pl.BlockSpec((2, _D), lambda i: (0, 0)),
            pl.BlockSpec(memory_space=pltpu.SMEM),
        ],
        out_specs=[
            pl.BlockSpec((_NBLK,), lambda i: (i,)),
            pl.BlockSpec((_NBLK,), lambda i: (i,)),
        ],
        out_shape=[
            jax.ShapeDtypeStruct((_N_PAD,), jnp.float32),
            jax.ShapeDtypeStruct((_N_PAD,), jnp.float32),
        ],
    )(node_embedding, w12t, b)
    return s, t


_sc_mesh = plsc.VectorSubcoreMesh(core_axis_name="c", subcore_axis_name="s")


@functools.partial(
    pl.kernel,
    mesh=_sc_mesh,
    out_type=jax.ShapeDtypeStruct((_N_EDGES,), jnp.float32),
    compiler_params=pltpu.CompilerParams(needs_layout_passes=False),
    scratch_types=[
        pltpu.VMEM((_N_PAD,), jnp.float32),
        pltpu.VMEM((_N_PAD,), jnp.float32),
        pltpu.VMEM((_E_PER_W,), jnp.int32),
        pltpu.VMEM((_E_PER_W,), jnp.int32),
        pltpu.VMEM((_E_PER_W,), jnp.float32),
        pltpu.SemaphoreType.DMA,
        pltpu.SemaphoreType.DMA,
    ],
)
def _sc_edge_logits(s_hbm, t_hbm, src_hbm, dst_hbm, out_hbm,
                    s_v, t_v, src_v, dst_v, o_v, sem, osem):
    wid = lax.axis_index("s") * _NC + lax.axis_index("c")
    base = wid * _E_PER_W
    nchunks = 5
    cw = _E_PER_W // nchunks

    c1 = pltpu.async_copy(s_hbm, s_v, sem)
    c2 = pltpu.async_copy(t_hbm, t_v, sem)
    copies = []
    for k in range(nchunks):
        sl = pl.ds(base + k * cw, cw)
        vl = pl.ds(k * cw, cw)
        copies.append((
            pltpu.async_copy(src_hbm.at[sl], src_v.at[vl], sem),
            pltpu.async_copy(dst_hbm.at[sl], dst_v.at[vl], sem),
        ))
    c1.wait()
    c2.wait()

    def body(i):
        sl = pl.ds(i, _L)
        gs = plsc.load_gather(s_v, [src_v[sl]])
        gt = plsc.load_gather(t_v, [dst_v[sl]])
        o_v[sl] = gs + gt

    for k in range(nchunks):
        copies[k][0].wait()
        copies[k][1].wait()
        plsc.parallel_loop(k * cw, (k + 1) * cw, _L, unroll=16)(body)
        pltpu.async_copy(o_v.at[pl.ds(k * cw, cw)],
                         out_hbm.at[pl.ds(base + k * cw, cw)], osem)
    for k in range(nchunks):
        pltpu.make_async_copy(o_v.at[pl.ds(k * cw, cw)],
                              out_hbm.at[pl.ds(base + k * cw, cw)],
                              osem).wait()


def kernel(node_embedding, edges, W, b):
    w12t = W.reshape(2, _D)
    src = edges[:, 0].astype(jnp.int32)
    dst = edges[:, 1].astype(jnp.int32)
    s, t = _make_tables(node_embedding, w12t, b)
    return _sc_edge_logits(s, t, src, dst).reshape(_N_EDGES, 1)


# speedup vs baseline: 1.0050x; 1.0050x over previous
import functools

import jax
import jax.numpy as jnp
from jax import lax
from jax.experimental import pallas as pl
from jax.experimental.pallas import tpu as pltpu
from jax.experimental.pallas import tpu_sc as plsc

_N_NODES = 10000
_N_EDGES = 320000
_D = 128

_info = plsc.get_sparse_core_info()
_NC = _info.num_cores
_NS = _info.num_subcores
_L = _info.num_lanes
_NW = _NC * _NS
_E_PER_W = _N_EDGES // _NW


_NBLK = 1024
_N_PAD = 10 * _NBLK


def _tc_tables_body(x_ref, w12t_ref, b_ref, s_ref, t_ref):
    r = lax.dot_general(
        w12t_ref[...], x_ref[...],
        dimension_numbers=(((1,), (1,)), ((), ())),
        preferred_element_type=jnp.float32,
    )
    s_ref[...] = r[0:1, :].reshape(_NBLK) + b_ref[0]
    t_ref[...] = r[1:2, :].reshape(_NBLK)


def _make_tables(node_embedding, w12t, b):
    s, t = pl.pallas_call(
        _tc_tables_body,
        grid=(_N_PAD // _NBLK,),
        in_specs=[
            pl.BlockSpec((_NBLK, _D), lambda i: (i, 0)),
            pl.BlockSpec((2, _D), lambda i: (0, 0)),
            pl.BlockSpec(memory_space=pltpu.SMEM),
        ],
        out_specs=[
            pl.BlockSpec((_NBLK,), lambda i: (i,)),
            pl.BlockSpec((_NBLK,), lambda i: (i,)),
        ],
        out_shape=[
            jax.ShapeDtypeStruct((_N_PAD,), jnp.float32),
            jax.ShapeDtypeStruct((_N_PAD,), jnp.float32),
        ],
    )(node_embedding, w12t, b)
    return s, t


_sc_mesh = plsc.VectorSubcoreMesh(core_axis_name="c", subcore_axis_name="s")


@functools.partial(
    pl.kernel,
    mesh=_sc_mesh,
    out_type=jax.ShapeDtypeStruct((_N_EDGES,), jnp.float32),
    compiler_params=pltpu.CompilerParams(needs_layout_passes=False),
    scratch_types=[
        pltpu.VMEM((_N_PAD,), jnp.float32),
        pltpu.VMEM((_N_PAD,), jnp.float32),
        pltpu.VMEM((_E_PER_W,), jnp.int32),
        pltpu.VMEM((_E_PER_W,), jnp.int32),
        pltpu.VMEM((_E_PER_W,), jnp.float32),
        pltpu.SemaphoreType.DMA,
        pltpu.SemaphoreType.DMA,
    ],
)
def _sc_edge_logits(s_hbm, t_hbm, src_hbm, dst_hbm, out_hbm,
                    s_v, t_v, src_v, dst_v, o_v, sem, osem):
    wid = lax.axis_index("s") * _NC + lax.axis_index("c")
    base = wid * _E_PER_W
    nchunks = 5
    cw = _E_PER_W // nchunks

    c1 = pltpu.async_copy(s_hbm, s_v, sem)
    c2 = pltpu.async_copy(t_hbm, t_v, sem)
    copies = []
    for k in range(nchunks):
        sl = pl.ds(base + k * cw, cw)
        vl = pl.ds(k * cw, cw)
        copies.append((
            pltpu.async_copy(src_hbm.at[sl], src_v.at[vl], sem),
            pltpu.async_copy(dst_hbm.at[sl], dst_v.at[vl], sem),
        ))
    c1.wait()
    c2.wait()

    def body(i):
        sl = pl.ds(i, _L)
        gs = plsc.load_gather(s_v, [src_v[sl]])
        gt = plsc.load_gather(t_v, [dst_v[sl]])
        o_v[sl] = gs + gt

    for k in range(nchunks):
        copies[k][0].wait()
        copies[k][1].wait()
        plsc.parallel_loop(k * cw, (k + 1) * cw, _L, unroll=8)(body)
        pltpu.async_copy(o_v.at[pl.ds(k * cw, cw)],
                         out_hbm.at[pl.ds(base + k * cw, cw)], osem)
    for k in range(nchunks):
        pltpu.make_async_copy(o_v.at[pl.ds(k * cw, cw)],
                              out_hbm.at[pl.ds(base + k * cw, cw)],
                              osem).wait()


def kernel(node_embedding, edges, W, b):
    w12t = W.reshape(2, _D)
    src = edges[:, 0].astype(jnp.int32)
    dst = edges[:, 1].astype(jnp.int32)
    s, t = _make_tables(node_embedding, w12t, b)
    return _sc_edge_logits(s, t, src, dst).reshape(_N_EDGES, 1)


# speedup vs baseline: 1.0559x; 1.0507x over previous
import functools

import jax
import jax.numpy as jnp
from jax import lax
from jax.experimental import pallas as pl
from jax.experimental.pallas import tpu as pltpu
from jax.experimental.pallas import tpu_sc as plsc

_N_NODES = 10000
_N_EDGES = 320000
_D = 128

_info = plsc.get_sparse_core_info()
_NC = _info.num_cores
_NS = _info.num_subcores
_L = _info.num_lanes
_NW = _NC * _NS
_E_PER_W = _N_EDGES // _NW


_NBLK = 2048
_N_PAD = 5 * _NBLK


def _tc_tables_body(x_ref, w12t_ref, b_ref, s_ref, t_ref):
    r = lax.dot_general(
        w12t_ref[...], x_ref[...],
        dimension_numbers=(((1,), (1,)), ((), ())),
        preferred_element_type=jnp.float32,
    )
    s_ref[...] = r[0:1, :].reshape(_NBLK) + b_ref[0]
    t_ref[...] = r[1:2, :].reshape(_NBLK)


def _make_tables(node_embedding, w12t, b):
    s, t = pl.pallas_call(
        _tc_tables_body,
        grid=(_N_PAD // _NBLK,),
        in_specs=[
            pl.BlockSpec((_NBLK, _D), lambda i: (i, 0)),
            pl.BlockSpec((2, _D), lambda i: (0, 0)),
            pl.BlockSpec(memory_space=pltpu.SMEM),
        ],
        out_specs=[
            pl.BlockSpec((_NBLK,), lambda i: (i,)),
            pl.BlockSpec((_NBLK,), lambda i: (i,)),
        ],
        out_shape=[
            jax.ShapeDtypeStruct((_N_PAD,), jnp.float32),
            jax.ShapeDtypeStruct((_N_PAD,), jnp.float32),
        ],
    )(node_embedding, w12t, b)
    return s, t


_sc_mesh = plsc.VectorSubcoreMesh(core_axis_name="c", subcore_axis_name="s")


@functools.partial(
    pl.kernel,
    mesh=_sc_mesh,
    out_type=jax.ShapeDtypeStruct((_N_EDGES,), jnp.float32),
    compiler_params=pltpu.CompilerParams(needs_layout_passes=False),
    scratch_types=[
        pltpu.VMEM((_N_PAD,), jnp.float32),
        pltpu.VMEM((_N_PAD,), jnp.float32),
        pltpu.VMEM((_E_PER_W,), jnp.int32),
        pltpu.VMEM((_E_PER_W,), jnp.int32),
        pltpu.VMEM((_E_PER_W,), jnp.float32),
        pltpu.SemaphoreType.DMA,
        pltpu.SemaphoreType.DMA,
    ],
)
def _sc_edge_logits(s_hbm, t_hbm, src_hbm, dst_hbm, out_hbm,
                    s_v, t_v, src_v, dst_v, o_v, sem, osem):
    wid = lax.axis_index("s") * _NC + lax.axis_index("c")
    base = wid * _E_PER_W
    nchunks = 5
    cw = _E_PER_W // nchunks

    c1 = pltpu.async_copy(s_hbm, s_v, sem)
    c2 = pltpu.async_copy(t_hbm, t_v, sem)
    copies = []
    for k in range(nchunks):
        sl = pl.ds(base + k * cw, cw)
        vl = pl.ds(k * cw, cw)
        copies.append((
            pltpu.async_copy(src_hbm.at[sl], src_v.at[vl], sem),
            pltpu.async_copy(dst_hbm.at[sl], dst_v.at[vl], sem),
        ))
    c1.wait()
    c2.wait()

    def body(i):
        sl = pl.ds(i, _L)
        gs = plsc.load_gather(s_v, [src_v[sl]])
        gt = plsc.load_gather(t_v, [dst_v[sl]])
        o_v[sl] = gs + gt

    for k in range(nchunks):
        copies[k][0].wait()
        copies[k][1].wait()
        plsc.parallel_loop(k * cw, (k + 1) * cw, _L, unroll=8)(body)
        pltpu.async_copy(o_v.at[pl.ds(k * cw, cw)],
                         out_hbm.at[pl.ds(base + k * cw, cw)], osem)
    for k in range(nchunks):
        pltpu.make_async_copy(o_v.at[pl.ds(k * cw, cw)],
                              out_hbm.at[pl.ds(base + k * cw, cw)],
                              osem).wait()


def kernel(node_embedding, edges, W, b):
    w12t = W.reshape(2, _D)
    src = edges[:, 0].astype(jnp.int32)
    dst = edges[:, 1].astype(jnp.int32)
    s, t = _make_tables(node_embedding, w12t, b)
    return _sc_edge_logits(s, t, src, dst).reshape(_N_EDGES, 1)


# speedup vs baseline: 1.0857x; 1.0282x over previous
import functools

import jax
import jax.numpy as jnp
from jax import lax
from jax.experimental import pallas as pl
from jax.experimental.pallas import tpu as pltpu
from jax.experimental.pallas import tpu_sc as plsc

_N_NODES = 10000
_N_EDGES = 320000
_D = 128

_info = plsc.get_sparse_core_info()
_NC = _info.num_cores
_NS = _info.num_subcores
_L = _info.num_lanes
_NW = _NC * _NS
_E_PER_W = _N_EDGES // _NW


_NBLK = 5120
_N_PAD = 2 * _NBLK


def _tc_tables_body(x_ref, w12t_ref, b_ref, s_ref, t_ref):
    r = lax.dot_general(
        w12t_ref[...], x_ref[...],
        dimension_numbers=(((1,), (1,)), ((), ())),
        preferred_element_type=jnp.float32,
    )
    s_ref[...] = r[0:1, :].reshape(_NBLK) + b_ref[0]
    t_ref[...] = r[1:2, :].reshape(_NBLK)


def _make_tables(node_embedding, w12t, b):
    s, t = pl.pallas_call(
        _tc_tables_body,
        grid=(_N_PAD // _NBLK,),
        in_specs=[
            pl.BlockSpec((_NBLK, _D), lambda i: (i, 0)),
            pl.BlockSpec((2, _D), lambda i: (0, 0)),
            pl.BlockSpec(memory_space=pltpu.SMEM),
        ],
        out_specs=[
            pl.BlockSpec((_NBLK,), lambda i: (i,)),
            pl.BlockSpec((_NBLK,), lambda i: (i,)),
        ],
        out_shape=[
            jax.ShapeDtypeStruct((_N_PAD,), jnp.float32),
            jax.ShapeDtypeStruct((_N_PAD,), jnp.float32),
        ],
    )(node_embedding, w12t, b)
    return s, t


_sc_mesh = plsc.VectorSubcoreMesh(core_axis_name="c", subcore_axis_name="s")


@functools.partial(
    pl.kernel,
    mesh=_sc_mesh,
    out_type=jax.ShapeDtypeStruct((_N_EDGES,), jnp.float32),
    compiler_params=pltpu.CompilerParams(needs_layout_passes=False),
    scratch_types=[
        pltpu.VMEM((_N_PAD,), jnp.float32),
        pltpu.VMEM((_N_PAD,), jnp.float32),
        pltpu.VMEM((_E_PER_W,), jnp.int32),
        pltpu.VMEM((_E_PER_W,), jnp.int32),
        pltpu.VMEM((_E_PER_W,), jnp.float32),
        pltpu.SemaphoreType.DMA,
        pltpu.SemaphoreType.DMA,
    ],
)
def _sc_edge_logits(s_hbm, t_hbm, src_hbm, dst_hbm, out_hbm,
                    s_v, t_v, src_v, dst_v, o_v, sem, osem):
    wid = lax.axis_index("s") * _NC + lax.axis_index("c")
    base = wid * _E_PER_W
    nchunks = 5
    cw = _E_PER_W // nchunks

    c1 = pltpu.async_copy(s_hbm, s_v, sem)
    c2 = pltpu.async_copy(t_hbm, t_v, sem)
    copies = []
    for k in range(nchunks):
        sl = pl.ds(base + k * cw, cw)
        vl = pl.ds(k * cw, cw)
        copies.append((
            pltpu.async_copy(src_hbm.at[sl], src_v.at[vl], sem),
            pltpu.async_copy(dst_hbm.at[sl], dst_v.at[vl], sem),
        ))
    c1.wait()
    c2.wait()

    def body(i):
        sl = pl.ds(i, _L)
        gs = plsc.load_gather(s_v, [src_v[sl]])
        gt = plsc.load_gather(t_v, [dst_v[sl]])
        o_v[sl] = gs + gt

    for k in range(nchunks):
        copies[k][0].wait()
        copies[k][1].wait()
        plsc.parallel_loop(k * cw, (k + 1) * cw, _L, unroll=8)(body)
        pltpu.async_copy(o_v.at[pl.ds(k * cw, cw)],
                         out_hbm.at[pl.ds(base + k * cw, cw)], osem)
    for k in range(nchunks):
        pltpu.make_async_copy(o_v.at[pl.ds(k * cw, cw)],
                              out_hbm.at[pl.ds(base + k * cw, cw)],
                              osem).wait()


def kernel(node_embedding, edges, W, b):
    w12t = W.reshape(2, _D)
    src = edges[:, 0].astype(jnp.int32)
    dst = edges[:, 1].astype(jnp.int32)
    s, t = _make_tables(node_embedding, w12t, b)
    return _sc_edge_logits(s, t, src, dst).reshape(_N_EDGES, 1)
